# baseline (speedup 1.0000x reference)
import functools

import jax
import jax.numpy as jnp
from jax import lax
from jax.experimental import pallas as pl
from jax.experimental.pallas import tpu as pltpu
from jax.experimental.pallas import tpu_sc as plsc

B = 1024
C = 20
D = 128
V = 100000
L = 16
NW = 32
BPW = B // NW
RPW = BPW * C
GCHUNK = 128
NCHUNK = RPW // GCHUNK


def _pool_body(xf_hbm, table_hbm, out_hbm, idx_v, rows_v, acc_v, sem):
    wid = lax.axis_index("s") * 2 + lax.axis_index("c")
    base = wid * BPW
    pltpu.sync_copy(xf_hbm.at[pl.ds(base * C, RPW)], idx_v)
    cps = [
        pltpu.async_copy(
            table_hbm.at[idx_v.at[pl.ds(c * GCHUNK, GCHUNK)]],
            rows_v.at[pl.ds(c * GCHUNK, GCHUNK)],
            sem,
        )
        for c in range(NCHUNK)
    ]
    for cp in cps:
        cp.wait()

    scale = jnp.float32(1.0 / C)

    def body(b, carry):
        row0 = b * C
        for d in range(D // L):
            acc = rows_v[row0, pl.ds(d * L, L)]
            for j in range(1, C):
                acc = acc + rows_v[row0 + j, pl.ds(d * L, L)]
            acc_v[b, pl.ds(d * L, L)] = acc * scale
        return carry

    lax.fori_loop(0, BPW, body, 0)
    pltpu.sync_copy(acc_v, out_hbm.at[pl.ds(base, BPW)])


def _pool(xf, table):
    mesh = plsc.VectorSubcoreMesh(core_axis_name="c", subcore_axis_name="s")
    return pl.kernel(
        _pool_body,
        out_type=jax.ShapeDtypeStruct((B, D), jnp.float32),
        mesh=mesh,
        scratch_types=[
            pltpu.VMEM((RPW,), jnp.int32),
            pltpu.VMEM((RPW, D), jnp.float32),
            pltpu.VMEM((BPW, D), jnp.float32),
            pltpu.SemaphoreType.DMA,
        ],
    )(xf, table)


BN = 2048
NB = (V + BN - 1) // BN


def _mm_body(h_ref, w_ref, b_ref, o_ref):
    h = h_ref[...].astype(jnp.bfloat16)
    w = w_ref[...].astype(jnp.bfloat16)
    acc = jnp.dot(h, w, preferred_element_type=jnp.float32)
    o_ref[...] = acc + b_ref[...]


def _matmul(h, lin_w, lin_b2):
    return pl.pallas_call(
        _mm_body,
        grid=(NB,),
        in_specs=[
            pl.BlockSpec((B, D), lambda i: (0, 0)),
            pl.BlockSpec((D, BN), lambda i: (0, i)),
            pl.BlockSpec((1, BN), lambda i: (0, i)),
        ],
        out_specs=pl.BlockSpec((B, BN), lambda i: (0, i)),
        out_shape=jax.ShapeDtypeStruct((B, V), jnp.float32),
    )(h, lin_w, lin_b2)


def kernel(x, embed_table, lin_w, lin_b):
    xf = x.reshape(-1).astype(jnp.int32)
    h = _pool(xf, embed_table)
    return _matmul(h, lin_w, lin_b.reshape(1, V))


# speedup vs baseline: 1.1416x; 1.1416x over previous
import functools

import jax
import jax.numpy as jnp
from jax import lax
from jax.experimental import pallas as pl
from jax.experimental.pallas import tpu as pltpu
from jax.experimental.pallas import tpu_sc as plsc

B = 1024
C = 20
D = 128
V = 100000
L = 16
NW = 32
BPW = B // NW
RPW = BPW * C
GCHUNK = 128
NCHUNK = RPW // GCHUNK


def _pool_body(xf_hbm, table_hbm, out_hbm, idx_v, rows_v, acc_v, sem):
    wid = lax.axis_index("s") * 2 + lax.axis_index("c")
    base = wid * BPW
    pltpu.sync_copy(xf_hbm.at[pl.ds(base * C, RPW)], idx_v)
    cps = [
        pltpu.async_copy(
            table_hbm.at[idx_v.at[pl.ds(c * GCHUNK, GCHUNK)]],
            rows_v.at[pl.ds(c * GCHUNK, GCHUNK)],
            sem,
        )
        for c in range(NCHUNK)
    ]
    for cp in cps:
        cp.wait()

    scale = jnp.float32(1.0 / C)

    def body(b, carry):
        row0 = b * C
        for d in range(D // L):
            acc = rows_v[row0, pl.ds(d * L, L)]
            for j in range(1, C):
                acc = acc + rows_v[row0 + j, pl.ds(d * L, L)]
            acc_v[b, pl.ds(d * L, L)] = acc * scale
        return carry

    lax.fori_loop(0, BPW, body, 0)
    pltpu.sync_copy(acc_v, out_hbm.at[pl.ds(base, BPW)])


def _pool(xf, table):
    mesh = plsc.VectorSubcoreMesh(core_axis_name="c", subcore_axis_name="s")
    return pl.kernel(
        _pool_body,
        out_type=jax.ShapeDtypeStruct((B, D), jnp.float32),
        mesh=mesh,
        scratch_types=[
            pltpu.VMEM((RPW,), jnp.int32),
            pltpu.VMEM((RPW, D), jnp.float32),
            pltpu.VMEM((BPW, D), jnp.float32),
            pltpu.SemaphoreType.DMA,
        ],
    )(xf, table)


BN = 2048
NB = 49
TAIL = 1664
VA = (NB - 1) * BN + TAIL
REM = V - VA
TCOL = V // 128
NBUF = 4
G = 8
RG = B // G


def _mm_body(h_ref, w_ref, b_ref, wt_ref, bt_ref, o_hbm, tail_ref, buf, sems):
    i = pl.program_id(0)
    s = lax.rem(i, NBUF)

    @pl.when(i >= NBUF)
    def _():
        for g in range(G):
            pltpu.make_async_copy(
                buf.at[s, pl.ds(g * RG, RG), :],
                o_hbm.at[pl.ds(g * RG, RG), pl.ds(0, BN)],
                sems.at[s],
            ).wait()

    h = h_ref[...].astype(jnp.bfloat16)
    w = w_ref[...].astype(jnp.bfloat16)
    acc = jnp.dot(h, w, preferred_element_type=jnp.float32)
    buf[s] = acc + b_ref[...]

    @pl.when(i < NB - 1)
    def _():
        for g in range(G):
            pltpu.make_async_copy(
                buf.at[s, pl.ds(g * RG, RG), :],
                o_hbm.at[pl.ds(g * RG, RG), pl.ds(i * BN, BN)],
                sems.at[s],
            ).start()

    @pl.when(i == NB - 1)
    def _():
        wt = wt_ref[...].astype(jnp.bfloat16)
        tail_ref[...] = (
            jnp.dot(h, wt, preferred_element_type=jnp.float32) + bt_ref[...]
        )
        for g in range(G):
            pltpu.make_async_copy(
                buf.at[s, pl.ds(g * RG, RG), pl.ds(0, TAIL)],
                o_hbm.at[pl.ds(g * RG, RG), pl.ds((NB - 1) * BN, TAIL)],
                sems.at[s],
            ).start()
        for k in range(NBUF - 1):
            sk = (NB - 2 - k) % NBUF
            for g in range(G):
                pltpu.make_async_copy(
                    buf.at[sk, pl.ds(g * RG, RG), :],
                    o_hbm.at[pl.ds(g * RG, RG), pl.ds(0, BN)],
                    sems.at[sk],
                ).wait()
        for g in range(G):
            pltpu.make_async_copy(
                buf.at[s, pl.ds(g * RG, RG), pl.ds(0, TAIL)],
                o_hbm.at[pl.ds(g * RG, RG), pl.ds(0, TAIL)],
                sems.at[s],
            ).wait()


def _matmul(h, lin_w, lin_b2):
    main, tail = pl.pallas_call(
        _mm_body,
        grid=(NB,),
        in_specs=[
            pl.BlockSpec((B, D), lambda i: (0, 0)),
            pl.BlockSpec((D, BN), lambda i: (0, i)),
            pl.BlockSpec((1, BN), lambda i: (0, i)),
            pl.BlockSpec((D, 128), lambda i: (0, TCOL)),
            pl.BlockSpec((1, 128), lambda i: (0, TCOL)),
        ],
        out_specs=[
            pl.BlockSpec(memory_space=pl.ANY),
            pl.BlockSpec((B, 128), lambda i: (0, 0)),
        ],
        out_shape=[
            jax.ShapeDtypeStruct((B, V), jnp.float32),
            jax.ShapeDtypeStruct((B, 128), jnp.float32),
        ],
        scratch_shapes=[
            pltpu.VMEM((NBUF, B, BN), jnp.float32),
            pltpu.SemaphoreType.DMA((NBUF,)),
        ],
    )(h, lin_w, lin_b2, lin_w, lin_b2)
    return lax.dynamic_update_slice(main, tail[:, :REM], (0, VA))


def kernel(x, embed_table, lin_w, lin_b):
    xf = x.reshape(-1).astype(jnp.int32)
    h = _pool(xf, embed_table)
    return _matmul(h, lin_w, lin_b.reshape(1, V))
